# baseline (speedup 1.0000x reference)
import functools

import jax
import jax.numpy as jnp
from jax import lax
from jax.experimental import pallas as pl
from jax.experimental.pallas import tpu as pltpu
from jax.experimental.pallas import tpu_sc as plsc

VOCAB = 1000000
EMB = 64
OUT = 128
B = 16384
L = 200

NC = 2
NS = 16
NW = NC * NS
BLK = 128
KPW = B // (NW * BLK)
NBUF = 2
HALVES = 2

_mesh = plsc.VectorSubcoreMesh(
    core_axis_name="c", subcore_axis_name="s", num_cores=NC, num_subcores=NS
)


@functools.partial(
    pl.kernel,
    out_type=jax.ShapeDtypeStruct((B, EMB), jnp.float32),
    mesh=_mesh,
    scratch_types=[
        pltpu.VMEM((BLK * L,), jnp.int32),
        pltpu.VMEM((NBUF, BLK), jnp.int32),
        pltpu.VMEM((NBUF, BLK, EMB), jnp.float32),
        pltpu.VMEM((BLK, EMB), jnp.float32),
        [pltpu.SemaphoreType.DMA] * NBUF,
    ],
    compiler_params=pltpu.CompilerParams(
        use_tc_tiling_on_sc=False, needs_layout_passes=False
    ),
)
def _pooled_sums(x_hbm, table_hbm, out_hbm, xv, idx_v, rows_v, accum_v, sems):
    wid = lax.axis_index("s") * NC + lax.axis_index("c")

    def _build_idx(bb, l):
        lane = lax.iota(jnp.int32, 16) * L + l
        for c in range(BLK // 16):
            vals = plsc.load_gather(xv, [lane + c * 16 * L])
            idx_v[bb, pl.ds(c * 16, 16)] = vals

    def _block(k, carry):
        blk = wid * KPW + k
        pltpu.sync_copy(x_hbm.at[pl.ds(blk * BLK * L, BLK * L)], xv)

        def _fire(bb):
            for h in range(HALVES):
                hw = BLK // HALVES
                pltpu.async_copy(
                    table_hbm.at[idx_v.at[bb, pl.ds(h * hw, hw)]],
                    rows_v.at[bb, pl.ds(h * hw, hw)],
                    sems[bb],
                )

        for bb in range(NBUF):
            _build_idx(bb, bb)
            _fire(bb)

        @plsc.parallel_loop(0, BLK, 1, unroll=8)
        def _zero(i):
            zero = jnp.zeros((16,), jnp.float32)
            for cc in range(EMB // 16):
                accum_v[i, pl.ds(cc * 16, 16)] = zero

        def _acc_slab(bb_rows):
            def _acc(i, c2):
                for cc in range(EMB // 16):
                    plsc.addupdate(
                        accum_v.at[i, pl.ds(cc * 16, 16)],
                        bb_rows[i, pl.ds(cc * 16, 16)],
                    )
                return c2

            lax.fori_loop(0, BLK, _acc, 0, unroll=8)

        def _step(m, carry2):
            for bb in range(NBUF):
                l = m * NBUF + bb
                pltpu.make_async_copy(
                    table_hbm.at[pl.ds(0, BLK)], rows_v.at[bb], sems[bb]
                ).wait()
                _acc_slab(rows_v.at[bb])
                l_next = jnp.minimum(l + NBUF, L - 1)
                _build_idx(bb, l_next)
                _fire(bb)
            return carry2

        lax.fori_loop(0, L // NBUF, _step, 0)

        for bb in range(NBUF):
            pltpu.make_async_copy(
                table_hbm.at[pl.ds(0, BLK)], rows_v.at[bb], sems[bb]
            ).wait()

        pltpu.sync_copy(accum_v, out_hbm.at[pl.ds(blk * BLK, BLK)])
        return carry

    lax.fori_loop(0, KPW, _block, 0)


def _project(pooled, W, b):
    BS = 1024

    def body(p_ref, w_ref, b_ref, o_ref):
        o_ref[...] = (
            jnp.dot(p_ref[...], w_ref[...], preferred_element_type=jnp.float32)
            * (1.0 / L)
            + b_ref[...]
        )

    return pl.pallas_call(
        body,
        grid=(B // BS,),
        in_specs=[
            pl.BlockSpec((BS, EMB), lambda i: (i, 0)),
            pl.BlockSpec((EMB, OUT), lambda i: (0, 0)),
            pl.BlockSpec((1, OUT), lambda i: (0, 0)),
        ],
        out_specs=pl.BlockSpec((BS, OUT), lambda i: (i, 0)),
        out_shape=jax.ShapeDtypeStruct((B, OUT), jnp.float32),
    )(pooled, W, b.reshape(1, OUT))


def kernel(x, table, W, b):
    x_flat = x.astype(jnp.int32).reshape(B * L)
    pooled = _pooled_sums(x_flat, table)
    return _project(pooled, W, b)


# speedup vs baseline: 1.0701x; 1.0701x over previous
import functools

import jax
import jax.numpy as jnp
from jax import lax
from jax.experimental import pallas as pl
from jax.experimental.pallas import tpu as pltpu
from jax.experimental.pallas import tpu_sc as plsc

VOCAB = 1000000
EMB = 64
OUT = 128
B = 16384
L = 200

NC = 2
NS = 16
NW = NC * NS
BLK = 128
KPW = B // (NW * BLK)
NBUF = 2
HALVES = 1

_mesh = plsc.VectorSubcoreMesh(
    core_axis_name="c", subcore_axis_name="s", num_cores=NC, num_subcores=NS
)


@functools.partial(
    pl.kernel,
    out_type=jax.ShapeDtypeStruct((B, EMB), jnp.float32),
    mesh=_mesh,
    scratch_types=[
        pltpu.VMEM((BLK * L,), jnp.int32),
        pltpu.VMEM((NBUF, BLK), jnp.int32),
        pltpu.VMEM((NBUF, BLK, EMB), jnp.float32),
        pltpu.VMEM((BLK, EMB), jnp.float32),
        [pltpu.SemaphoreType.DMA] * NBUF,
    ],
    compiler_params=pltpu.CompilerParams(
        use_tc_tiling_on_sc=False, needs_layout_passes=False
    ),
)
def _pooled_sums(x_hbm, table_hbm, out_hbm, xv, idx_v, rows_v, accum_v, sems):
    wid = lax.axis_index("s") * NC + lax.axis_index("c")

    def _build_idx(bb, l):
        lane = lax.iota(jnp.int32, 16) * L + l
        for c in range(BLK // 16):
            vals = plsc.load_gather(xv, [lane + c * 16 * L])
            idx_v[bb, pl.ds(c * 16, 16)] = vals

    def _block(k, carry):
        blk = wid * KPW + k
        pltpu.sync_copy(x_hbm.at[pl.ds(blk * BLK * L, BLK * L)], xv)

        def _fire(bb):
            for h in range(HALVES):
                hw = BLK // HALVES
                pltpu.async_copy(
                    table_hbm.at[idx_v.at[bb, pl.ds(h * hw, hw)]],
                    rows_v.at[bb, pl.ds(h * hw, hw)],
                    sems[bb],
                )

        for bb in range(NBUF):
            _build_idx(bb, bb)
            _fire(bb)

        @plsc.parallel_loop(0, BLK, 1, unroll=8)
        def _zero(i):
            zero = jnp.zeros((16,), jnp.float32)
            for cc in range(EMB // 16):
                accum_v[i, pl.ds(cc * 16, 16)] = zero

        def _acc_slab(bb_rows):
            def _acc(i, c2):
                for cc in range(EMB // 16):
                    plsc.addupdate(
                        accum_v.at[i, pl.ds(cc * 16, 16)],
                        bb_rows[i, pl.ds(cc * 16, 16)],
                    )
                return c2

            lax.fori_loop(0, BLK, _acc, 0, unroll=8)

        def _step(m, carry2):
            for bb in range(NBUF):
                l = m * NBUF + bb
                pltpu.make_async_copy(
                    table_hbm.at[pl.ds(0, BLK)], rows_v.at[bb], sems[bb]
                ).wait()
                _acc_slab(rows_v.at[bb])
                l_next = jnp.minimum(l + NBUF, L - 1)
                _build_idx(bb, l_next)
                _fire(bb)
            return carry2

        lax.fori_loop(0, L // NBUF, _step, 0)

        for bb in range(NBUF):
            pltpu.make_async_copy(
                table_hbm.at[pl.ds(0, BLK)], rows_v.at[bb], sems[bb]
            ).wait()

        pltpu.sync_copy(accum_v, out_hbm.at[pl.ds(blk * BLK, BLK)])
        return carry

    lax.fori_loop(0, KPW, _block, 0)


def _project(pooled, W, b):
    BS = 1024

    def body(p_ref, w_ref, b_ref, o_ref):
        o_ref[...] = (
            jnp.dot(p_ref[...], w_ref[...], preferred_element_type=jnp.float32)
            * (1.0 / L)
            + b_ref[...]
        )

    return pl.pallas_call(
        body,
        grid=(B // BS,),
        in_specs=[
            pl.BlockSpec((BS, EMB), lambda i: (i, 0)),
            pl.BlockSpec((EMB, OUT), lambda i: (0, 0)),
            pl.BlockSpec((1, OUT), lambda i: (0, 0)),
        ],
        out_specs=pl.BlockSpec((BS, OUT), lambda i: (i, 0)),
        out_shape=jax.ShapeDtypeStruct((B, OUT), jnp.float32),
    )(pooled, W, b.reshape(1, OUT))


def kernel(x, table, W, b):
    x_flat = x.astype(jnp.int32).reshape(B * L)
    pooled = _pooled_sums(x_flat, table)
    return _project(pooled, W, b)
